# baseline (speedup 1.0000x reference)
import functools

import jax
import jax.numpy as jnp
from jax import lax
from jax.experimental import pallas as pl
from jax.experimental.pallas import tpu as pltpu
from jax.experimental.pallas import tpu_sc as plsc

_EXT = 10.0
_NPOS = 441
_NENT = 4
_B, _S, _D = 4, 256, 64
_NPAIR = _B * _S

_info = plsc.get_sparse_core_info()
_NC, _NS, _NL = _info.num_cores, _info.num_subcores, _info.num_lanes
_NW = _NC * _NS
_ROWS_PER_W = _NPAIR // _NW
_W_PER_B = _NW // _B


def _bucket(d):
    a = jnp.minimum(jnp.abs(d), jnp.float32(_EXT))
    a5 = a + jnp.float32(0.5)
    ti = a5.astype(jnp.int32)
    tie = ti.astype(jnp.float32) == a5
    odd = jnp.bitwise_and(ti, 1) == 1
    r = ti - jnp.where(jnp.logical_and(tie, odd), 1, 0)
    r = jnp.where(d < jnp.float32(0.0), -r, r)
    return r + 10


def _sc_gather_body(with_entity, nb, refs):
    if with_entity:
        (px_hbm, py_hbm, et_hbm, tab_hbm, out_hbm,
         tab_sh, px_v, py_v, vt_v, idx_v, rows, sem_g, sem_o) = refs
    else:
        (px_hbm, py_hbm, tab_hbm, out_hbm,
         tab_sh, px_v, py_v, idx_v, rows, sem_g, sem_o) = refs
    cid = lax.axis_index("c")
    sid = lax.axis_index("s")
    wid = sid * _NC + cid
    w_per_b = _NW // nb
    rows_per_w = (nb * _S) // _NW
    b = wid // w_per_b
    i_base = (wid % w_per_b) * rows_per_w

    @pl.when(sid == 0)
    def _stage():
        pltpu.sync_copy(tab_hbm, tab_sh)

    pltpu.sync_copy(px_hbm.at[b], px_v)
    pltpu.sync_copy(py_hbm.at[b], py_v)
    if with_entity:
        pltpu.sync_copy(et_hbm.at[b], vt_v)
        for jj in range(_S // _NL):
            vt_v[jj] = vt_v[jj] * _NPOS

    plsc.subcore_barrier()

    dnums = lax.GatherDimensionNumbers(
        offset_dims=(), collapsed_slice_dims=(0,), start_index_map=(0,))

    def compute_idx(p, s):
        i = i_base + p
        u = jnp.where(i < _S // 2, 2 * i, 2 * i - (_S - 1))
        l = u % _NL
        qx = px_v[u // _NL]
        qy = py_v[u // _NL]
        li = jnp.full((_NL, 1), l, jnp.int32)
        xi = lax.gather(qx, li, dnums, (1,),
                        mode=lax.GatherScatterMode.PROMISE_IN_BOUNDS)
        yi = lax.gather(qy, li, dnums, (1,),
                        mode=lax.GatherScatterMode.PROMISE_IN_BOUNDS)
        for jj in range(_S // _NL):
            idx = _bucket(px_v[jj] - xi) + 21 * _bucket(py_v[jj] - yi)
            if with_entity:
                idx = idx + vt_v[jj]
            h, o = divmod(jj * _NL, 128)
            idx_v[s, h, pl.ds(o, _NL)] = idx

    def gather_copies(s):
        return [
            pltpu.make_async_copy(
                tab_sh.at[idx_v.at[s].at[h]],
                rows.at[s].at[pl.ds(h * 128, 128)], sem_g)
            for h in range(2)
        ]

    def out_copies(p, s):
        base = (b * _S + (i_base + p)) * _S
        return [pltpu.make_async_copy(
            rows.at[s], out_hbm.at[pl.ds(base, _S)], sem_o)]

    compute_idx(0, 0)
    for c in gather_copies(0):
        c.start()

    def step(g, carry):
        for s in range(2):
            p = 2 * g + s
            sn = 1 - s
            for c in gather_copies(s):
                c.wait()

            @pl.when(p >= 1)
            def _wait_prev_out():
                for c in out_copies(p - 1, sn):
                    c.wait()

            for c in out_copies(p, s):
                c.start()

            @pl.when(p + 1 < rows_per_w)
            def _prefetch_next():
                compute_idx(p + 1, sn)
                for c in gather_copies(sn):
                    c.start()
        return carry

    lax.fori_loop(0, rows_per_w // 2, step, 0)
    for c in out_copies(rows_per_w - 1, 1):
        c.wait()


def _make_sc_call(with_entity, vocab, nb=_B):
    mesh = plsc.VectorSubcoreMesh(core_axis_name="c", subcore_axis_name="s")
    f32 = jnp.float32
    scratch = [
        pltpu.VMEM_SHARED((vocab, _D), f32),
        pltpu.VMEM((_S // _NL, _NL), f32),
        pltpu.VMEM((_S // _NL, _NL), f32),
    ]
    if with_entity:
        scratch.append(pltpu.VMEM((_S // _NL, _NL), jnp.int32))
    scratch += [
        pltpu.VMEM((2, 2, 128), jnp.int32),
        pltpu.VMEM((2, _S, _D), f32),
        pltpu.SemaphoreType.DMA,
        pltpu.SemaphoreType.DMA,
    ]
    return pl.kernel(
        lambda *refs: _sc_gather_body(with_entity, nb, refs),
        mesh=mesh,
        compiler_params=pltpu.CompilerParams(use_tc_tiling_on_sc=False),
        out_type=jax.ShapeDtypeStruct((nb * _S * _S, _D), f32),
        scratch_types=scratch,
    )


_TG = 128


def _make_tc_tx_body(tg):
    def body(x_ref, y_ref):
        x3 = x_ref[...].reshape(tg, 128, 128)
        y_ref[:, :, : _S // 2] = jnp.swapaxes(x3[:, :, :_D], 1, 2)
        y_ref[:, :, _S // 2:] = jnp.swapaxes(x3[:, :, _D:], 1, 2)
    return body


def _tc_transpose(x):
    return pl.pallas_call(
        _make_tc_tx_body(_TG),
        grid=(_NPAIR // _TG,),
        in_specs=[pl.BlockSpec((_TG * 128, 128), lambda p: (p, 0))],
        out_specs=pl.BlockSpec((_TG, _D, _S), lambda p: (p, 0, 0)),
        out_shape=jax.ShapeDtypeStruct((_NPAIR, _D, _S), jnp.float32),
    )(x)


_TGH = 64


def _tc_transpose_chain(x1, x2):
    half = _NPAIR // 2
    n = half // _TGH
    body = _make_tc_tx_body(_TGH)

    def body2(_buf_ref, x_ref, y_ref):
        body(x_ref, y_ref)

    buf = pl.pallas_call(
        body,
        grid=(n,),
        in_specs=[pl.BlockSpec((_TGH * 128, 128), lambda p: (p, 0))],
        out_specs=pl.BlockSpec((_TGH, _D, _S), lambda p: (p, 0, 0)),
        out_shape=jax.ShapeDtypeStruct((_NPAIR, _D, _S), jnp.float32),
    )(x1)
    return pl.pallas_call(
        body2,
        grid=(n,),
        in_specs=[
            pl.BlockSpec(memory_space=pl.ANY),
            pl.BlockSpec((_TGH * 128, 128), lambda p: (p, 0)),
        ],
        out_specs=pl.BlockSpec((_TGH, _D, _S), lambda p: (p + n, 0, 0)),
        out_shape=jax.ShapeDtypeStruct((_NPAIR, _D, _S), jnp.float32),
        input_output_aliases={0: 0},
    )(buf, x2)


def _interleave_j(a):
    return jnp.stack([a[..., : _S // 2], a[..., _S // 2:]], axis=-1).reshape(
        *a.shape[:-1], _S)


@jax.jit
def _run(positions, entity_type, keys_table, values_table):
    px = _interleave_j(positions[..., 0]).reshape(_B, _S // _NL, _NL)
    py = _interleave_j(positions[..., 1]).reshape(_B, _S // _NL, _NL)
    et = _interleave_j(entity_type.astype(jnp.int32)).reshape(_B, _S // _NL, _NL)
    hb = _B // 2
    sck = _make_sc_call(False, _NPOS, nb=hb)
    outk1 = sck(px[:hb], py[:hb], keys_table)
    outk2 = sck(px[hb:], py[hb:], keys_table)
    outv = _make_sc_call(True, _NPOS * _NENT)(px, py, et, values_table)
    tk = _tc_transpose_chain(outk1.reshape(_NPAIR * 64, 128),
                             outk2.reshape(_NPAIR * 64, 128))
    tv = _tc_transpose(outv.reshape(_NPAIR * 128, 128))
    tk = tk.reshape(_B, _S, _D, _S).transpose(0, 1, 3, 2)
    tv = tv.reshape(_B, _S, _D, _S).transpose(0, 1, 3, 2)
    return (tk, tv)


def kernel(positions, entity_type, keys_table, values_table):
    return _run(positions, entity_type, keys_table, values_table)


# speedup vs baseline: 1.0206x; 1.0206x over previous
import functools

import jax
import jax.numpy as jnp
from jax import lax
from jax.experimental import pallas as pl
from jax.experimental.pallas import tpu as pltpu
from jax.experimental.pallas import tpu_sc as plsc

_EXT = 10.0
_NPOS = 441
_NENT = 4
_B, _S, _D = 4, 256, 64
_NPAIR = _B * _S

_info = plsc.get_sparse_core_info()
_NC, _NS, _NL = _info.num_cores, _info.num_subcores, _info.num_lanes
_NW = _NC * _NS
_ROWS_PER_W = _NPAIR // _NW
_W_PER_B = _NW // _B


def _bucket(d):
    a = jnp.minimum(jnp.abs(d), jnp.float32(_EXT))
    a5 = a + jnp.float32(0.5)
    ti = a5.astype(jnp.int32)
    tie = ti.astype(jnp.float32) == a5
    odd = jnp.bitwise_and(ti, 1) == 1
    r = ti - jnp.where(jnp.logical_and(tie, odd), 1, 0)
    r = jnp.where(d < jnp.float32(0.0), -r, r)
    return r + 10


def _sc_gather_body(with_entity, refs):
    if with_entity:
        (px_hbm, py_hbm, et_hbm, tab_hbm, out_hbm,
         tab_sh, px_v, py_v, vt_v, idx_v, rows, sem_g, sem_o) = refs
    else:
        (px_hbm, py_hbm, tab_hbm, out_hbm,
         tab_sh, px_v, py_v, idx_v, rows, sem_g, sem_o) = refs
    cid = lax.axis_index("c")
    sid = lax.axis_index("s")
    wid = sid * _NC + cid
    b = wid // _W_PER_B
    i_base = (wid % _W_PER_B) * _ROWS_PER_W

    @pl.when(sid == 0)
    def _stage():
        pltpu.sync_copy(tab_hbm, tab_sh)

    pltpu.sync_copy(px_hbm.at[b], px_v)
    pltpu.sync_copy(py_hbm.at[b], py_v)
    if with_entity:
        pltpu.sync_copy(et_hbm.at[b], vt_v)
        for jj in range(_S // _NL):
            vt_v[jj] = vt_v[jj] * _NPOS

    plsc.subcore_barrier()

    dnums = lax.GatherDimensionNumbers(
        offset_dims=(), collapsed_slice_dims=(0,), start_index_map=(0,))

    def compute_idx(p, s):
        i = i_base + p
        u = jnp.where(i < _S // 2, 2 * i, 2 * i - (_S - 1))
        l = u % _NL
        qx = px_v[u // _NL]
        qy = py_v[u // _NL]
        li = jnp.full((_NL, 1), l, jnp.int32)
        xi = lax.gather(qx, li, dnums, (1,),
                        mode=lax.GatherScatterMode.PROMISE_IN_BOUNDS)
        yi = lax.gather(qy, li, dnums, (1,),
                        mode=lax.GatherScatterMode.PROMISE_IN_BOUNDS)
        for jj in range(_S // _NL):
            idx = _bucket(px_v[jj] - xi) + 21 * _bucket(py_v[jj] - yi)
            if with_entity:
                idx = idx + vt_v[jj]
            h, o = divmod(jj * _NL, 128)
            idx_v[s, h, pl.ds(o, _NL)] = idx

    def gather_copies(s):
        return [
            pltpu.make_async_copy(
                tab_sh.at[idx_v.at[s].at[h]],
                rows.at[s].at[pl.ds(h * 128, 128)], sem_g)
            for h in range(2)
        ]

    def out_copies(p, s):
        base = (b * _S + (i_base + p)) * _S
        return [pltpu.make_async_copy(
            rows.at[s], out_hbm.at[pl.ds(base, _S)], sem_o)]

    compute_idx(0, 0)
    for c in gather_copies(0):
        c.start()

    def step(g, carry):
        for s in range(2):
            p = 2 * g + s
            sn = 1 - s
            for c in gather_copies(s):
                c.wait()

            @pl.when(p >= 1)
            def _wait_prev_out():
                for c in out_copies(p - 1, sn):
                    c.wait()

            for c in out_copies(p, s):
                c.start()

            @pl.when(p + 1 < _ROWS_PER_W)
            def _prefetch_next():
                compute_idx(p + 1, sn)
                for c in gather_copies(sn):
                    c.start()
        return carry

    lax.fori_loop(0, _ROWS_PER_W // 2, step, 0)
    for c in out_copies(_ROWS_PER_W - 1, 1):
        c.wait()


def _make_sc_call(with_entity, vocab):
    mesh = plsc.VectorSubcoreMesh(core_axis_name="c", subcore_axis_name="s")
    f32 = jnp.float32
    scratch = [
        pltpu.VMEM_SHARED((vocab, _D), f32),
        pltpu.VMEM((_S // _NL, _NL), f32),
        pltpu.VMEM((_S // _NL, _NL), f32),
    ]
    if with_entity:
        scratch.append(pltpu.VMEM((_S // _NL, _NL), jnp.int32))
    scratch += [
        pltpu.VMEM((2, 2, 128), jnp.int32),
        pltpu.VMEM((2, _S, _D), f32),
        pltpu.SemaphoreType.DMA,
        pltpu.SemaphoreType.DMA,
    ]
    return pl.kernel(
        lambda *refs: _sc_gather_body(with_entity, refs),
        mesh=mesh,
        compiler_params=pltpu.CompilerParams(use_tc_tiling_on_sc=False),
        out_type=jax.ShapeDtypeStruct((_NPAIR * _S, _D), f32),
        scratch_types=scratch,
    )


_TG = 128


def _tc_tx_body(x_ref, y_ref):
    x3 = x_ref[...].reshape(_TG, 128, 128)
    y_ref[:, :, : _S // 2] = jnp.swapaxes(x3[:, :, :_D], 1, 2)
    y_ref[:, :, _S // 2:] = jnp.swapaxes(x3[:, :, _D:], 1, 2)


def _tc_transpose(x):
    return pl.pallas_call(
        _tc_tx_body,
        grid=(_NPAIR // _TG,),
        in_specs=[pl.BlockSpec((_TG * 128, 128), lambda p: (p, 0))],
        out_specs=pl.BlockSpec((_TG, _D, _S), lambda p: (p, 0, 0)),
        out_shape=jax.ShapeDtypeStruct((_NPAIR, _D, _S), jnp.float32),
    )(x)


def _interleave_j(a):
    return jnp.stack([a[..., : _S // 2], a[..., _S // 2:]], axis=-1).reshape(
        *a.shape[:-1], _S)


@jax.jit
def _run(positions, entity_type, keys_table, values_table):
    px = _interleave_j(positions[..., 0]).reshape(_B, _S // _NL, _NL)
    py = _interleave_j(positions[..., 1]).reshape(_B, _S // _NL, _NL)
    et = _interleave_j(entity_type.astype(jnp.int32)).reshape(_B, _S // _NL, _NL)
    outk = _make_sc_call(False, _NPOS)(px, py, keys_table)
    outv = _make_sc_call(True, _NPOS * _NENT)(px, py, et, values_table)
    tk = _tc_transpose(outk.reshape(_NPAIR * 128, 128))
    tv = _tc_transpose(outv.reshape(_NPAIR * 128, 128))
    tk = tk.reshape(_B, _S, _D, _S).transpose(0, 1, 3, 2)
    tv = tv.reshape(_B, _S, _D, _S).transpose(0, 1, 3, 2)
    return (tk, tv)


def kernel(positions, entity_type, keys_table, values_table):
    return _run(positions, entity_type, keys_table, values_table)
